# Initial kernel scaffold; baseline (speedup 1.0000x reference)
import jax
import jax.numpy as jnp
from jax.experimental import pallas as pl


def kernel(indices, species):
    raise NotImplementedError("write your pallas kernel here")


# speedup vs baseline: 6.5812x; 6.5812x over previous
import functools

import jax
import jax.numpy as jnp
from jax import lax
from jax.experimental import pallas as pl
from jax.experimental.pallas import tpu as pltpu
from jax.experimental.pallas import tpu_sc as plsc

_BATCH = 16384
_HIST = 200
_VOCAB = 1000
_EMB = 128

_TOTAL = _BATCH * _HIST
_IDXW = 128
_NROWS = _TOTAL // _IDXW
_NW = 32
_ROWS_PER_W = _NROWS // _NW
_K = 5
_CHUNK = _K * _IDXW
_ITERS = _ROWS_PER_W // _K


def _sc_gather(idx2d, table):
    mesh = plsc.VectorSubcoreMesh(core_axis_name="c", subcore_axis_name="s")
    nc = mesh.num_cores

    @functools.partial(
        pl.kernel,
        out_type=jax.ShapeDtypeStruct((_TOTAL, _EMB), jnp.float32),
        mesh=mesh,
        scratch_types=[
            pltpu.VMEM((_CHUNK,), jnp.int32),
            pltpu.VMEM((_CHUNK, _EMB), jnp.float32),
            pltpu.SemaphoreType.DMA,
        ],
    )
    def body(table_hbm, idx_hbm, out_hbm, idx_v, rows_v, sem):
        wid = lax.axis_index("s") * nc + lax.axis_index("c")
        idx_base = wid * _ROWS_PER_W * _IDXW

        def one_chunk(i, _):
            cb = idx_base + i * _CHUNK
            pltpu.sync_copy(idx_hbm.at[pl.ds(cb, _CHUNK)], idx_v)
            copies = []
            for j in range(_K):
                copies.append(
                    pltpu.async_copy(
                        table_hbm.at[idx_v.at[pl.ds(j * _IDXW, _IDXW)]],
                        rows_v.at[pl.ds(j * _IDXW, _IDXW)],
                        sem,
                    )
                )
            for c in copies:
                c.wait()
            pltpu.sync_copy(rows_v, out_hbm.at[pl.ds(cb, _CHUNK)])
            return 0

        lax.fori_loop(0, _ITERS, one_chunk, 0)

    return body(table, idx2d)


def kernel(indices, species):
    idx1d = indices.reshape(_TOTAL).astype(jnp.int32)
    out = _sc_gather(idx1d, species)
    return out.reshape(_BATCH, _HIST, _EMB)


# speedup vs baseline: 6.5969x; 1.0024x over previous
import functools

import jax
import jax.numpy as jnp
from jax import lax
from jax.experimental import pallas as pl
from jax.experimental.pallas import tpu as pltpu
from jax.experimental.pallas import tpu_sc as plsc

_BATCH = 16384
_HIST = 200
_VOCAB = 1000
_EMB = 128

_TOTAL = _BATCH * _HIST
_IDXW = 128
_NW = 32
_PER_W = _TOTAL // _NW

_IB = 25600
_NBLK = _PER_W // _IB
_CHUNK = 2 * _IDXW
_CPB = _IB // _CHUNK
_PAIRS = _CPB // 2


def _sc_gather(idx1d, table):
    mesh = plsc.VectorSubcoreMesh(core_axis_name="c", subcore_axis_name="s")
    nc = mesh.num_cores

    @functools.partial(
        pl.kernel,
        out_type=jax.ShapeDtypeStruct((_TOTAL, _EMB), jnp.float32),
        mesh=mesh,
        scratch_types=[
            pltpu.VMEM((_IB,), jnp.int32),
            pltpu.VMEM((2 * _CHUNK, _EMB), jnp.float32),
            pltpu.SemaphoreType.DMA,
            pltpu.SemaphoreType.DMA,
            pltpu.SemaphoreType.DMA,
            pltpu.SemaphoreType.DMA,
        ],
    )
    def body(table_hbm, idx_hbm, out_hbm, idx_v, rows_v, g0, g1, s0, s1):
        wid = lax.axis_index("s") * nc + lax.axis_index("c")
        base = wid * _PER_W

        def gathers(idx_off, buf, sem):
            for j in range(2):
                pltpu.async_copy(
                    table_hbm.at[idx_v.at[pl.ds(idx_off + j * _IDXW, _IDXW)]],
                    rows_v.at[pl.ds(buf * _CHUNK + j * _IDXW, _IDXW)],
                    sem,
                )

        def wait_g(buf, sem):
            pltpu.make_async_copy(
                out_hbm.at[pl.ds(0, _CHUNK)],
                rows_v.at[pl.ds(buf * _CHUNK, _CHUNK)],
                sem,
            ).wait()

        def store(out_off, buf, sem):
            pltpu.async_copy(
                rows_v.at[pl.ds(buf * _CHUNK, _CHUNK)],
                out_hbm.at[pl.ds(out_off, _CHUNK)],
                sem,
            )

        def wait_s(buf, sem):
            pltpu.make_async_copy(
                rows_v.at[pl.ds(buf * _CHUNK, _CHUNK)],
                out_hbm.at[pl.ds(0, _CHUNK)],
                sem,
            ).wait()

        def block_body(blk, _):
            bbase = base + blk * _IB
            pltpu.sync_copy(idx_hbm.at[pl.ds(bbase, _IB)], idx_v)
            gathers(0, 0, g0)
            gathers(_CHUNK, 1, g1)

            def pair(g, _):
                off0 = 2 * g * _CHUNK
                wait_g(0, g0)
                store(bbase + off0, 0, s0)
                wait_g(1, g1)
                store(bbase + off0 + _CHUNK, 1, s1)
                wait_s(0, s0)
                gathers(off0 + 2 * _CHUNK, 0, g0)
                wait_s(1, s1)
                gathers(off0 + 3 * _CHUNK, 1, g1)
                return 0

            lax.fori_loop(0, _PAIRS - 1, pair, 0)

            off = (_CPB - 2) * _CHUNK
            wait_g(0, g0)
            store(bbase + off, 0, s0)
            wait_g(1, g1)
            store(bbase + off + _CHUNK, 1, s1)
            wait_s(0, s0)
            wait_s(1, s1)
            return 0

        lax.fori_loop(0, _NBLK, block_body, 0)

    return body(table, idx1d)


def kernel(indices, species):
    idx1d = indices.reshape(_TOTAL).astype(jnp.int32)
    out = _sc_gather(idx1d, species)
    return out.reshape(_BATCH, _HIST, _EMB)


# speedup vs baseline: 12.8377x; 1.9460x over previous
import functools

import jax
import jax.numpy as jnp
from jax import lax
from jax.experimental import pallas as pl
from jax.experimental.pallas import tpu as pltpu
from jax.experimental.pallas import tpu_sc as plsc

_BATCH = 16384
_HIST = 200
_VOCAB = 1000
_EMB = 128

_TOTAL = _BATCH * _HIST
_IDXW = 128
_NW = 32
_PER_W = _TOTAL // _NW

_IB = 25600
_NBLK = _PER_W // _IB
_CHUNK = 2 * _IDXW
_CPB = _IB // _CHUNK
_PAIRS = _CPB // 2


def _sc_gather(idx1d, table):
    mesh = plsc.VectorSubcoreMesh(core_axis_name="c", subcore_axis_name="s")
    nc = mesh.num_cores

    @functools.partial(
        pl.kernel,
        out_type=jax.ShapeDtypeStruct((_TOTAL, _EMB), jnp.float32),
        mesh=mesh,
        scratch_types=[
            pltpu.VMEM((_IB,), jnp.int32),
            pltpu.VMEM((2 * _CHUNK, _EMB), jnp.float32),
            pltpu.VMEM_SHARED((_VOCAB, _EMB), jnp.float32),
            pltpu.SemaphoreType.DMA,
            pltpu.SemaphoreType.DMA,
            pltpu.SemaphoreType.DMA,
            pltpu.SemaphoreType.DMA,
        ],
    )
    def body(table_hbm, idx_hbm, out_hbm, idx_v, rows_v, table_sp, g0, g1, s0, s1):
        cid = lax.axis_index("c")
        sid = lax.axis_index("s")
        wid = sid * nc + cid
        base = wid * _PER_W

        @pl.when(sid == 0)
        def _():
            pltpu.sync_copy(table_hbm, table_sp)

        plsc.subcore_barrier()

        def gathers(idx_off, buf, sem):
            for j in range(2):
                pltpu.async_copy(
                    table_sp.at[idx_v.at[pl.ds(idx_off + j * _IDXW, _IDXW)]],
                    rows_v.at[pl.ds(buf * _CHUNK + j * _IDXW, _IDXW)],
                    sem,
                )

        def wait_g(buf, sem):
            pltpu.make_async_copy(
                out_hbm.at[pl.ds(0, _CHUNK)],
                rows_v.at[pl.ds(buf * _CHUNK, _CHUNK)],
                sem,
            ).wait()

        def store(out_off, buf, sem):
            pltpu.async_copy(
                rows_v.at[pl.ds(buf * _CHUNK, _CHUNK)],
                out_hbm.at[pl.ds(out_off, _CHUNK)],
                sem,
            )

        def wait_s(buf, sem):
            pltpu.make_async_copy(
                rows_v.at[pl.ds(buf * _CHUNK, _CHUNK)],
                out_hbm.at[pl.ds(0, _CHUNK)],
                sem,
            ).wait()

        def block_body(blk, _):
            bbase = base + blk * _IB
            pltpu.sync_copy(idx_hbm.at[pl.ds(bbase, _IB)], idx_v)
            gathers(0, 0, g0)
            gathers(_CHUNK, 1, g1)

            def pair(g, _):
                off0 = 2 * g * _CHUNK
                wait_g(0, g0)
                store(bbase + off0, 0, s0)
                wait_g(1, g1)
                store(bbase + off0 + _CHUNK, 1, s1)
                wait_s(0, s0)
                gathers(off0 + 2 * _CHUNK, 0, g0)
                wait_s(1, s1)
                gathers(off0 + 3 * _CHUNK, 1, g1)
                return 0

            lax.fori_loop(0, _PAIRS - 1, pair, 0)

            off = (_CPB - 2) * _CHUNK
            wait_g(0, g0)
            store(bbase + off, 0, s0)
            wait_g(1, g1)
            store(bbase + off + _CHUNK, 1, s1)
            wait_s(0, s0)
            wait_s(1, s1)
            return 0

        lax.fori_loop(0, _NBLK, block_body, 0)

    return body(table, idx1d)


def kernel(indices, species):
    idx1d = indices.reshape(_TOTAL).astype(jnp.int32)
    out = _sc_gather(idx1d, species)
    return out.reshape(_BATCH, _HIST, _EMB)


# speedup vs baseline: 19.1682x; 1.4931x over previous
import functools

import jax
import jax.numpy as jnp
from jax import lax
from jax.experimental import pallas as pl
from jax.experimental.pallas import tpu as pltpu
from jax.experimental.pallas import tpu_sc as plsc

_BATCH = 16384
_HIST = 200
_VOCAB = 1000
_EMB = 128

_TOTAL = _BATCH * _HIST
_CHUNK = 128
_NBUF = 4
_GRP = _NBUF * _CHUNK
_NW = 32
_PER_W = _TOTAL // _NW

_IB = 25600
_NBLK = _PER_W // _IB
_GPB = _IB // _GRP


def _sc_gather(idx1d, table):
    mesh = plsc.VectorSubcoreMesh(core_axis_name="c", subcore_axis_name="s")
    nc = mesh.num_cores

    @functools.partial(
        pl.kernel,
        out_type=jax.ShapeDtypeStruct((_TOTAL, _EMB), jnp.float32),
        mesh=mesh,
        scratch_types=[
            pltpu.VMEM((_IB,), jnp.int32),
            pltpu.VMEM((_NBUF * _CHUNK, _EMB), jnp.float32),
            pltpu.VMEM_SHARED((_VOCAB, _EMB), jnp.float32),
            [pltpu.SemaphoreType.DMA] * _NBUF,
            [pltpu.SemaphoreType.DMA] * _NBUF,
        ],
    )
    def body(table_hbm, idx_hbm, out_hbm, idx_v, rows_v, table_sp, gsems, ssems):
        cid = lax.axis_index("c")
        sid = lax.axis_index("s")
        wid = sid * nc + cid
        base = wid * _PER_W

        @pl.when(sid == 0)
        def _():
            pltpu.sync_copy(table_hbm, table_sp)

        plsc.subcore_barrier()

        def gather(idx_off, buf):
            pltpu.async_copy(
                table_sp.at[idx_v.at[pl.ds(idx_off, _CHUNK)]],
                rows_v.at[pl.ds(buf * _CHUNK, _CHUNK)],
                gsems[buf],
            )

        def wait_g(buf):
            pltpu.make_async_copy(
                out_hbm.at[pl.ds(0, _CHUNK)],
                rows_v.at[pl.ds(buf * _CHUNK, _CHUNK)],
                gsems[buf],
            ).wait()

        def store(out_off, buf):
            pltpu.async_copy(
                rows_v.at[pl.ds(buf * _CHUNK, _CHUNK)],
                out_hbm.at[pl.ds(out_off, _CHUNK)],
                ssems[buf],
            )

        def wait_s(buf):
            pltpu.make_async_copy(
                rows_v.at[pl.ds(buf * _CHUNK, _CHUNK)],
                out_hbm.at[pl.ds(0, _CHUNK)],
                ssems[buf],
            ).wait()

        def block_body(blk, _):
            bbase = base + blk * _IB
            pltpu.sync_copy(idx_hbm.at[pl.ds(bbase, _IB)], idx_v)
            for b in range(_NBUF):
                gather(b * _CHUNK, b)

            def group(g, _):
                off = g * _GRP
                for b in range(_NBUF):
                    wait_g(b)
                    store(bbase + off + b * _CHUNK, b)
                for b in range(_NBUF):
                    wait_s(b)
                    gather(off + _GRP + b * _CHUNK, b)
                return 0

            lax.fori_loop(0, _GPB - 1, group, 0)

            off = (_GPB - 1) * _GRP
            for b in range(_NBUF):
                wait_g(b)
                store(bbase + off + b * _CHUNK, b)
            for b in range(_NBUF):
                wait_s(b)
            return 0

        lax.fori_loop(0, _NBLK, block_body, 0)

    return body(table, idx1d)


def kernel(indices, species):
    idx1d = indices.reshape(_TOTAL).astype(jnp.int32)
    out = _sc_gather(idx1d, species)
    return out.reshape(_BATCH, _HIST, _EMB)


# speedup vs baseline: 19.4889x; 1.0167x over previous
import functools

import jax
import jax.numpy as jnp
from jax import lax
from jax.experimental import pallas as pl
from jax.experimental.pallas import tpu as pltpu
from jax.experimental.pallas import tpu_sc as plsc

_BATCH = 16384
_HIST = 200
_VOCAB = 1000
_EMB = 128

_TOTAL = _BATCH * _HIST
_CHUNK = 80
_NBUF = 8
_LAG = 4
_NW = 32
_PER_W = _TOTAL // _NW

_IB = 25600
_NBLK = _PER_W // _IB
_CPB = _IB // _CHUNK
_MAIN = _CPB - _NBUF
_BODIES = _MAIN // _NBUF


def _sc_gather(idx1d, table):
    mesh = plsc.VectorSubcoreMesh(core_axis_name="c", subcore_axis_name="s")
    nc = mesh.num_cores

    @functools.partial(
        pl.kernel,
        out_type=jax.ShapeDtypeStruct((_TOTAL, _EMB), jnp.float32),
        mesh=mesh,
        scratch_types=[
            pltpu.VMEM((_IB,), jnp.int32),
            pltpu.VMEM((_NBUF * _CHUNK, _EMB), jnp.float32),
            pltpu.VMEM_SHARED((_VOCAB, _EMB), jnp.float32),
            [pltpu.SemaphoreType.DMA] * _NBUF,
            [pltpu.SemaphoreType.DMA] * _NBUF,
        ],
    )
    def body(table_hbm, idx_hbm, out_hbm, idx_v, rows_v, table_sp, gsems, ssems):
        cid = lax.axis_index("c")
        sid = lax.axis_index("s")
        wid = sid * nc + cid
        base = wid * _PER_W

        @pl.when(sid == 0)
        def _():
            pltpu.sync_copy(table_hbm, table_sp)

        plsc.subcore_barrier()

        def gather(idx_off, buf):
            pltpu.async_copy(
                table_sp.at[idx_v.at[pl.ds(idx_off, _CHUNK)]],
                rows_v.at[pl.ds(buf * _CHUNK, _CHUNK)],
                gsems[buf],
            )

        def wait_g(buf):
            pltpu.make_async_copy(
                out_hbm.at[pl.ds(0, _CHUNK)],
                rows_v.at[pl.ds(buf * _CHUNK, _CHUNK)],
                gsems[buf],
            ).wait()

        def store(out_off, buf):
            pltpu.async_copy(
                rows_v.at[pl.ds(buf * _CHUNK, _CHUNK)],
                out_hbm.at[pl.ds(out_off, _CHUNK)],
                ssems[buf],
            )

        def wait_s(buf):
            pltpu.make_async_copy(
                rows_v.at[pl.ds(buf * _CHUNK, _CHUNK)],
                out_hbm.at[pl.ds(0, _CHUNK)],
                ssems[buf],
            ).wait()

        def block_body(blk, _):
            bbase = base + blk * _IB
            pltpu.sync_copy(idx_hbm.at[pl.ds(bbase, _IB)], idx_v)

            for b in range(_NBUF):
                gather(b * _CHUNK, b)
            for b in range(_LAG):
                wait_g(b)
                store(bbase + b * _CHUNK, b)

            def group(g, _):
                c0 = _NBUF + g * _NBUF
                for k in range(_NBUF):
                    c = c0 + k
                    buf = k
                    wait_s(buf)
                    gather(c * _CHUNK, buf)
                    sbuf = (k + _NBUF - _LAG) % _NBUF
                    wait_g(sbuf)
                    store(bbase + (c - _LAG) * _CHUNK, sbuf)
                return 0

            lax.fori_loop(0, _BODIES, group, 0)

            for k in range(_LAG):
                c = _CPB - _LAG + k
                buf = c % _NBUF
                wait_g(buf)
                store(bbase + c * _CHUNK, buf)
            for b in range(_NBUF):
                wait_s(b)
            return 0

        lax.fori_loop(0, _NBLK, block_body, 0)

    return body(table, idx1d)


def kernel(indices, species):
    idx1d = indices.reshape(_TOTAL).astype(jnp.int32)
    out = _sc_gather(idx1d, species)
    return out.reshape(_BATCH, _HIST, _EMB)


# speedup vs baseline: 19.7327x; 1.0125x over previous
import functools

import jax
import jax.numpy as jnp
from jax import lax
from jax.experimental import pallas as pl
from jax.experimental.pallas import tpu as pltpu
from jax.experimental.pallas import tpu_sc as plsc

_BATCH = 16384
_HIST = 200
_VOCAB = 1000
_EMB = 128

_TOTAL = _BATCH * _HIST
_CHUNK = 80
_NBUF = 8
_LAG = 4
_NW = 32
_PER_W = _TOTAL // _NW

_HB = 10240
_CPH = _HB // _CHUNK
_NHALF = _PER_W // _HB
_NCHUNK = _PER_W // _CHUNK
_BODIES = (_NCHUNK - _NBUF) // _NBUF
_BPH = _CPH // _NBUF


def _sc_gather(idx1d, table):
    mesh = plsc.VectorSubcoreMesh(core_axis_name="c", subcore_axis_name="s")
    nc = mesh.num_cores

    @functools.partial(
        pl.kernel,
        out_type=jax.ShapeDtypeStruct((_TOTAL, _EMB), jnp.float32),
        mesh=mesh,
        scratch_types=[
            pltpu.VMEM((2 * _HB,), jnp.int32),
            pltpu.VMEM((_NBUF * _CHUNK, _EMB), jnp.float32),
            pltpu.VMEM_SHARED((_VOCAB, _EMB), jnp.float32),
            [pltpu.SemaphoreType.DMA] * _NBUF,
            [pltpu.SemaphoreType.DMA] * _NBUF,
            pltpu.SemaphoreType.DMA,
        ],
    )
    def body(table_hbm, idx_hbm, out_hbm, idx_v, rows_v, table_sp,
             gsems, ssems, isem):
        cid = lax.axis_index("c")
        sid = lax.axis_index("s")
        wid = sid * nc + cid
        base = wid * _PER_W

        @pl.when(sid == 0)
        def _():
            pltpu.sync_copy(table_hbm, table_sp)

        plsc.subcore_barrier()

        def idx_off(c):
            return ((c >> 7) & 1) * _HB + (c & (_CPH - 1)) * _CHUNK

        def gather(c, buf):
            pltpu.async_copy(
                table_sp.at[idx_v.at[pl.ds(idx_off(c), _CHUNK)]],
                rows_v.at[pl.ds(buf * _CHUNK, _CHUNK)],
                gsems[buf],
            )

        def wait_g(buf):
            pltpu.make_async_copy(
                out_hbm.at[pl.ds(0, _CHUNK)],
                rows_v.at[pl.ds(buf * _CHUNK, _CHUNK)],
                gsems[buf],
            ).wait()

        def store(c, buf):
            pltpu.async_copy(
                rows_v.at[pl.ds(buf * _CHUNK, _CHUNK)],
                out_hbm.at[pl.ds(base + c * _CHUNK, _CHUNK)],
                ssems[buf],
            )

        def wait_s(buf):
            pltpu.make_async_copy(
                rows_v.at[pl.ds(buf * _CHUNK, _CHUNK)],
                out_hbm.at[pl.ds(0, _CHUNK)],
                ssems[buf],
            ).wait()

        def prefetch(h):
            pltpu.async_copy(
                idx_hbm.at[pl.ds(base + h * _HB, _HB)],
                idx_v.at[pl.ds((h & 1) * _HB, _HB)],
                isem,
            )

        def wait_prefetch():
            pltpu.make_async_copy(
                idx_hbm.at[pl.ds(0, _HB)],
                idx_v.at[pl.ds(0, _HB)],
                isem,
            ).wait()

        pltpu.sync_copy(idx_hbm.at[pl.ds(base, _HB)], idx_v.at[pl.ds(0, _HB)])
        prefetch(1)
        for b in range(_NBUF):
            gather(b, b)
        for b in range(_LAG):
            wait_g(b)
            store(b, b)

        def group(g, _):
            c0 = _NBUF + g * _NBUF

            @pl.when(g % _BPH == _BPH - 1)
            def _():
                wait_prefetch()

            for k in range(_NBUF):
                c = c0 + k
                buf = k
                wait_s(buf)
                gather(c, buf)
                sbuf = (k + _NBUF - _LAG) % _NBUF
                wait_g(sbuf)
                store(c - _LAG, sbuf)

            m = g // _BPH
            @pl.when(jnp.logical_and(g % _BPH == 0,
                                     jnp.logical_and(g >= _BPH,
                                                     m <= _NHALF - 2)))
            def _():
                pltpu.async_copy(
                    idx_hbm.at[pl.ds(base + (m + 1) * _HB, _HB)],
                    idx_v.at[pl.ds(((m + 1) & 1) * _HB, _HB)],
                    isem,
                )
            return 0

        lax.fori_loop(0, _BODIES, group, 0)

        for k in range(_LAG):
            c = _NCHUNK - _LAG + k
            buf = c % _NBUF
            wait_g(buf)
            store(c, buf)
        for b in range(_NBUF):
            wait_s(b)

    return body(table, idx1d)


def kernel(indices, species):
    idx1d = indices.reshape(_TOTAL).astype(jnp.int32)
    out = _sc_gather(idx1d, species)
    return out.reshape(_BATCH, _HIST, _EMB)


# speedup vs baseline: 19.7404x; 1.0004x over previous
import functools

import jax
import jax.numpy as jnp
from jax import lax
from jax.experimental import pallas as pl
from jax.experimental.pallas import tpu as pltpu
from jax.experimental.pallas import tpu_sc as plsc

_BATCH = 16384
_HIST = 200
_VOCAB = 1000
_EMB = 128

_TOTAL = _BATCH * _HIST
_CHUNK = 80
_NBUF = 8
_NPAIR = 4
_PROWS = 2 * _CHUNK
_NW = 32
_PER_W = _TOTAL // _NW

_HB = 10240
_CPH = _HB // _CHUNK
_NHALF = _PER_W // _HB
_NCHUNK = _PER_W // _CHUNK
_BODIES = (_NCHUNK - _NBUF) // _NBUF
_BPH = _CPH // _NBUF


def _sc_gather(idx1d, table):
    mesh = plsc.VectorSubcoreMesh(core_axis_name="c", subcore_axis_name="s")
    nc = mesh.num_cores

    @functools.partial(
        pl.kernel,
        out_type=jax.ShapeDtypeStruct((_TOTAL, _EMB), jnp.float32),
        mesh=mesh,
        scratch_types=[
            pltpu.VMEM((2 * _HB,), jnp.int32),
            pltpu.VMEM((_NBUF * _CHUNK, _EMB), jnp.float32),
            pltpu.VMEM_SHARED((_VOCAB, _EMB), jnp.float32),
            [pltpu.SemaphoreType.DMA] * _NBUF,
            [pltpu.SemaphoreType.DMA] * _NPAIR,
            pltpu.SemaphoreType.DMA,
        ],
    )
    def body(table_hbm, idx_hbm, out_hbm, idx_v, rows_v, table_sp,
             gsems, ssems, isem):
        cid = lax.axis_index("c")
        sid = lax.axis_index("s")
        wid = sid * nc + cid
        base = wid * _PER_W

        @pl.when(sid == 0)
        def _():
            pltpu.sync_copy(table_hbm, table_sp)

        plsc.subcore_barrier()

        def idx_off(c):
            return ((c >> 7) & 1) * _HB + (c & (_CPH - 1)) * _CHUNK

        def gather(c, buf):
            pltpu.async_copy(
                table_sp.at[idx_v.at[pl.ds(idx_off(c), _CHUNK)]],
                rows_v.at[pl.ds(buf * _CHUNK, _CHUNK)],
                gsems[buf],
            )

        def wait_g(buf):
            pltpu.make_async_copy(
                out_hbm.at[pl.ds(0, _CHUNK)],
                rows_v.at[pl.ds(buf * _CHUNK, _CHUNK)],
                gsems[buf],
            ).wait()

        def store2(q, pair):
            pltpu.async_copy(
                rows_v.at[pl.ds(pair * _PROWS, _PROWS)],
                out_hbm.at[pl.ds(base + q * _PROWS, _PROWS)],
                ssems[pair],
            )

        def wait_s2(pair):
            pltpu.make_async_copy(
                rows_v.at[pl.ds(pair * _PROWS, _PROWS)],
                out_hbm.at[pl.ds(0, _PROWS)],
                ssems[pair],
            ).wait()

        def prefetch(h):
            pltpu.async_copy(
                idx_hbm.at[pl.ds(base + h * _HB, _HB)],
                idx_v.at[pl.ds((h & 1) * _HB, _HB)],
                isem,
            )

        def wait_prefetch():
            pltpu.make_async_copy(
                idx_hbm.at[pl.ds(0, _HB)],
                idx_v.at[pl.ds(0, _HB)],
                isem,
            ).wait()

        pltpu.sync_copy(idx_hbm.at[pl.ds(base, _HB)], idx_v.at[pl.ds(0, _HB)])
        prefetch(1)
        for b in range(_NBUF):
            gather(b, b)
        for p in range(2):
            wait_g(2 * p)
            wait_g(2 * p + 1)
            store2(p, p)

        def group(g, _):
            q0 = _NPAIR + g * _NPAIR

            @pl.when(g % _BPH == _BPH - 1)
            def _():
                wait_prefetch()

            for k in range(_NPAIR):
                q = q0 + k
                wait_s2(k)
                gather(2 * q, 2 * k)
                gather(2 * q + 1, 2 * k + 1)
                spair = (k + _NPAIR - 2) % _NPAIR
                wait_g(2 * spair)
                wait_g(2 * spair + 1)
                store2(q - 2, spair)

            m = g // _BPH
            @pl.when(jnp.logical_and(g % _BPH == 0,
                                     jnp.logical_and(g >= _BPH,
                                                     m <= _NHALF - 2)))
            def _():
                pltpu.async_copy(
                    idx_hbm.at[pl.ds(base + (m + 1) * _HB, _HB)],
                    idx_v.at[pl.ds(((m + 1) & 1) * _HB, _HB)],
                    isem,
                )
            return 0

        lax.fori_loop(0, _BODIES, group, 0)

        npairs = _NCHUNK // 2
        for j in range(2):
            q = npairs - 2 + j
            pair = q % _NPAIR
            wait_g(2 * pair)
            wait_g(2 * pair + 1)
            store2(q, pair)
        for p in range(_NPAIR):
            wait_s2(p)

    return body(table, idx1d)


def kernel(indices, species):
    idx1d = indices.reshape(_TOTAL).astype(jnp.int32)
    out = _sc_gather(idx1d, species)
    return out.reshape(_BATCH, _HIST, _EMB)


# speedup vs baseline: 19.7866x; 1.0023x over previous
import functools

import jax
import jax.numpy as jnp
from jax import lax
from jax.experimental import pallas as pl
from jax.experimental.pallas import tpu as pltpu
from jax.experimental.pallas import tpu_sc as plsc

_BATCH = 16384
_HIST = 200
_VOCAB = 1000
_EMB = 128

_TOTAL = _BATCH * _HIST
_CHUNK = 80
_NBUF = 8
_NPAIR = 4
_PROWS = 2 * _CHUNK
_NW = 32
_PER_W = _TOTAL // _NW

_HB = 10240
_CPH = _HB // _CHUNK
_NHALF = _PER_W // _HB
_NCHUNK = _PER_W // _CHUNK
_BODIES = (_NCHUNK - _NBUF) // _NBUF
_BPH = _CPH // _NBUF


def _sc_gather(idx1d, table):
    mesh = plsc.VectorSubcoreMesh(core_axis_name="c", subcore_axis_name="s")
    nc = mesh.num_cores

    @functools.partial(
        pl.kernel,
        out_type=jax.ShapeDtypeStruct((_TOTAL, _EMB), jnp.float32),
        mesh=mesh,
        scratch_types=[
            pltpu.VMEM((2 * _HB,), jnp.int32),
            pltpu.VMEM((_NBUF * _CHUNK, _EMB), jnp.float32),
            pltpu.VMEM_SHARED((_VOCAB, _EMB), jnp.float32),
            [pltpu.SemaphoreType.DMA] * _NBUF,
            [pltpu.SemaphoreType.DMA] * _NPAIR,
            pltpu.SemaphoreType.DMA,
        ],
    )
    def body(table_hbm, idx_hbm, out_hbm, idx_v, rows_v, table_sp,
             gsems, ssems, isem):
        cid = lax.axis_index("c")
        sid = lax.axis_index("s")
        wid = sid * nc + cid
        base = wid * _PER_W

        @pl.when(sid == 0)
        def _():
            pltpu.sync_copy(table_hbm, table_sp)

        plsc.subcore_barrier()

        def idx_off(c):
            return ((c >> 7) & 1) * _HB + (c & (_CPH - 1)) * _CHUNK

        def gather(c, buf):
            pltpu.async_copy(
                table_sp.at[idx_v.at[pl.ds(idx_off(c), _CHUNK)]],
                rows_v.at[pl.ds(buf * _CHUNK, _CHUNK)],
                gsems[buf],
            )

        def wait_g(buf):
            pltpu.make_async_copy(
                out_hbm.at[pl.ds(0, _CHUNK)],
                rows_v.at[pl.ds(buf * _CHUNK, _CHUNK)],
                gsems[buf],
            ).wait()

        def store2(q, pair):
            pltpu.async_copy(
                rows_v.at[pl.ds(pair * _PROWS, _PROWS)],
                out_hbm.at[pl.ds(base + q * _PROWS, _PROWS)],
                ssems[pair],
            )

        def wait_s2(pair):
            pltpu.make_async_copy(
                rows_v.at[pl.ds(pair * _PROWS, _PROWS)],
                out_hbm.at[pl.ds(0, _PROWS)],
                ssems[pair],
            ).wait()

        def prefetch(h):
            pltpu.async_copy(
                idx_hbm.at[pl.ds(base + h * _HB, _HB)],
                idx_v.at[pl.ds((h & 1) * _HB, _HB)],
                isem,
            )

        def wait_prefetch():
            pltpu.make_async_copy(
                idx_hbm.at[pl.ds(0, _HB)],
                idx_v.at[pl.ds(0, _HB)],
                isem,
            ).wait()

        pltpu.sync_copy(idx_hbm.at[pl.ds(base, _HB)], idx_v.at[pl.ds(0, _HB)])
        prefetch(1)
        for b in range(_NBUF):
            gather(b, b)
        for p in range(2):
            wait_g(2 * p)
            wait_g(2 * p + 1)
            store2(p, p)

        def group(g, _):
            q0 = _NPAIR + g * _NPAIR

            @pl.when(g % _BPH == _BPH - 1)
            def _():
                wait_prefetch()

            for k in range(_NPAIR):
                q = q0 + k
                spair = (k + _NPAIR - 2) % _NPAIR
                wait_g(2 * spair)
                wait_g(2 * spair + 1)
                store2(q - 2, spair)
                wait_s2(k)
                gather(2 * q, 2 * k)
                gather(2 * q + 1, 2 * k + 1)

            m = g // _BPH
            @pl.when(jnp.logical_and(g % _BPH == 0,
                                     jnp.logical_and(g >= _BPH,
                                                     m <= _NHALF - 2)))
            def _():
                pltpu.async_copy(
                    idx_hbm.at[pl.ds(base + (m + 1) * _HB, _HB)],
                    idx_v.at[pl.ds(((m + 1) & 1) * _HB, _HB)],
                    isem,
                )
            return 0

        lax.fori_loop(0, _BODIES, group, 0)

        npairs = _NCHUNK // 2
        for j in range(2):
            q = npairs - 2 + j
            pair = q % _NPAIR
            wait_g(2 * pair)
            wait_g(2 * pair + 1)
            store2(q, pair)
        for p in range(_NPAIR):
            wait_s2(p)

    return body(table, idx1d)


def kernel(indices, species):
    idx1d = indices.reshape(_TOTAL).astype(jnp.int32)
    out = _sc_gather(idx1d, species)
    return out.reshape(_BATCH, _HIST, _EMB)
